# baseline (speedup 1.0000x reference)
import functools

import jax
import jax.numpy as jnp
from jax import lax
from jax.experimental import pallas as pl
from jax.experimental.pallas import tpu as pltpu
from jax.experimental.pallas import tpu_sc as plsc

NC, NS = 2, 16
NW = NC * NS
L = 16

O, S, D = 26, 100000, 32
NU = 16384
NQ_O, NQ_S = 26, 4096
NQ = NQ_O * NQ_S
TPW = 81920
TPAD = NW * TPW
QPW = NQ // NW
QCH = 128
NCH = QPW // QCH
ZROW = NU
NZPAD = 2048

_mesh = plsc.VectorSubcoreMesh(core_axis_name="c", subcore_axis_name="s")
_params = pltpu.CompilerParams(needs_layout_passes=False,
                               use_tc_tiling_on_sc=False)


def _worker_id():
    return lax.axis_index("s") * NC + lax.axis_index("c")


RPW = NU // NW
RCH = 256
ZPW = NZPAD // NW


def _table_body(org_hbm, samp_hbm, inp_hbm, tbl_hbm, pad_hbm,
                org_v, samp_v, tbl_v, bounce_v, sem):
    wid = _worker_id()
    lo = wid * TPW
    rbase = wid * RPW

    d1 = pltpu.async_copy(org_hbm, org_v, sem)
    d2 = pltpu.async_copy(samp_hbm, samp_v, sem)
    din = pltpu.async_copy(inp_hbm.at[pl.ds(rbase, RCH)], bounce_v, sem)

    zero = jnp.zeros((L,), jnp.int32)

    def z_body(i, c):
        base = i * (8 * L)
        for u in range(8):
            tbl_v[pl.ds(base + u * L, L)] = zero
        return c

    lax.fori_loop(0, TPW // (8 * L), z_body, 0)

    din.wait()
    pltpu.sync_copy(bounce_v, pad_hbm.at[pl.ds(rbase, RCH)])
    din = pltpu.async_copy(inp_hbm.at[pl.ds(rbase + RCH, RCH)], bounce_v, sem)

    d1.wait()
    d2.wait()

    iota = lax.iota(jnp.int32, L)

    def u_body(i, c):
        o = org_v[pl.ds(i * L, L)]
        sp = samp_v[pl.ds(i * L, L)]
        local = o * S + sp - lo
        m = (local >= 0) & (local < TPW)
        val = i * L + iota + 1
        plsc.store_scatter(tbl_v, [local], val, mask=m)
        return c

    lax.fori_loop(0, NU // L, u_body, 0)

    din.wait()
    pltpu.sync_copy(bounce_v, pad_hbm.at[pl.ds(rbase + RCH, RCH)])

    zf = jnp.zeros((L,), jnp.float32)

    def zr_body(i, c):
        bounce_v[i >> 1, pl.ds((i & 1) * L, L)] = zf
        return c

    lax.fori_loop(0, ZPW * D // L, zr_body, 0)
    pltpu.sync_copy(bounce_v.at[pl.ds(0, ZPW)],
                    pad_hbm.at[pl.ds(NU + wid * ZPW, ZPW)])

    pltpu.sync_copy(tbl_v, tbl_hbm.at[pl.ds(lo, TPW)])


@jax.jit
def _build_table(org_ids, samp_ids, inp):
    return pl.kernel(
        _table_body,
        out_type=(jax.ShapeDtypeStruct((TPAD,), jnp.int32),
                  jax.ShapeDtypeStruct((NU + NZPAD, D), jnp.float32)),
        mesh=_mesh,
        compiler_params=_params,
        scratch_types=[
            pltpu.VMEM((NU,), jnp.int32),
            pltpu.VMEM((NU,), jnp.int32),
            pltpu.VMEM((TPW,), jnp.int32),
            pltpu.VMEM((RCH, D), jnp.float32),
            pltpu.SemaphoreType.DMA,
        ],
    )(org_ids, samp_ids, inp)


def _gather_body(org_hbm, samp_hbm, tbl_hbm, inp_hbm, out_hbm,
                 org_v, samp_v, keys_v, jv_v, sel_v, rows_v, sem):
    wid = _worker_id()
    qbase = wid * QPW

    d1 = pltpu.async_copy(org_hbm, org_v, sem)
    d2 = pltpu.async_copy(samp_hbm, samp_v, sem)
    d1.wait()
    d2.wait()

    iota = lax.iota(jnp.int32, L)

    def k_body(i, c):
        q = qbase + i * L + iota
        o = q >> 12
        s = q & (NQ_S - 1)
        og = plsc.load_gather(org_v, [o])
        sg = plsc.load_gather(samp_v, [s])
        keys_v[i >> 3, pl.ds((i & 7) * L, L)] = og * S + sg
        return c

    lax.fori_loop(0, QPW // L, k_body, 0)

    descs = [
        pltpu.async_copy(tbl_hbm.at[keys_v.at[c]], jv_v.at[c], sem)
        for c in range(NCH)
    ]
    for d in descs:
        d.wait()

    def s_body(i, c):
        v = jv_v[i >> 3, pl.ds((i & 7) * L, L)]
        q = qbase + i * L + iota
        miss = ZROW + (q & (NZPAD - 1))
        sel_v[i >> 3, pl.ds((i & 7) * L, L)] = jnp.where(v > 0, v - 1, miss)
        return c

    lax.fori_loop(0, QPW // L, s_body, 0)

    descs = [
        pltpu.async_copy(inp_hbm.at[sel_v.at[c]],
                         rows_v.at[pl.ds(c * QCH, QCH)], sem)
        for c in range(NCH)
    ]
    for d in descs:
        d.wait()

    pltpu.sync_copy(rows_v, out_hbm.at[pl.ds(qbase, QPW)])


@jax.jit
def _join_gather(org_pad, samp_ids, table, inp_pad):
    return pl.kernel(
        _gather_body,
        out_type=jax.ShapeDtypeStruct((NQ, D), jnp.float32),
        mesh=_mesh,
        compiler_params=_params,
        scratch_types=[
            pltpu.VMEM((2 * L,), jnp.int32),
            pltpu.VMEM((NQ_S,), jnp.int32),
            pltpu.VMEM((NCH, QCH), jnp.int32),
            pltpu.VMEM((NCH, QCH), jnp.int32),
            pltpu.VMEM((NCH, QCH), jnp.int32),
            pltpu.VMEM((QPW, D), jnp.float32),
            pltpu.SemaphoreType.DMA,
        ],
    )(org_pad, samp_ids, table, inp_pad)


def kernel(buffer, sample_id, organization_id, input, get_sample_id, get_org_id):
    del buffer
    org_ids = organization_id.astype(jnp.int32)
    samp_ids = sample_id.astype(jnp.int32)
    org_pad = jnp.concatenate(
        [get_org_id.astype(jnp.int32), jnp.zeros((2 * L - NQ_O,), jnp.int32)])
    table, inp_pad = _build_table(org_ids, samp_ids, input)
    out = _join_gather(org_pad, get_sample_id.astype(jnp.int32), table, inp_pad)
    return out.reshape(NQ_O, NQ_S, D)


# speedup vs baseline: 1.0020x; 1.0020x over previous
import functools

import jax
import jax.numpy as jnp
from jax import lax
from jax.experimental import pallas as pl
from jax.experimental.pallas import tpu as pltpu
from jax.experimental.pallas import tpu_sc as plsc

NC, NS = 2, 16
NW = NC * NS
L = 16

O, S, D = 26, 100000, 32
NU = 16384
NQ_O, NQ_S = 26, 4096
NQ = NQ_O * NQ_S
TPW = 81920
TPAD = NW * TPW
QPW = NQ // NW
QCH = 128
NCH = QPW // QCH
ZROW = NU
NZPAD = 2048

_mesh = plsc.VectorSubcoreMesh(core_axis_name="c", subcore_axis_name="s")
_params = pltpu.CompilerParams(needs_layout_passes=False,
                               use_tc_tiling_on_sc=False)


def _worker_id():
    return lax.axis_index("s") * NC + lax.axis_index("c")


RPW = NU // NW
RCH = 256
ZPW = NZPAD // NW


def _table_body(org_hbm, samp_hbm, inp_hbm, tbl_hbm, pad_hbm,
                org_v, samp_v, tbl_v, bounce_v, sem, sem_in):
    wid = _worker_id()
    lo = wid * TPW
    rbase = wid * RPW

    d1 = pltpu.async_copy(org_hbm, org_v, sem)
    d2 = pltpu.async_copy(samp_hbm, samp_v, sem)
    din = pltpu.async_copy(inp_hbm.at[pl.ds(rbase, RCH)], bounce_v, sem_in)

    zero = jnp.zeros((L,), jnp.int32)

    def z_body(i, c):
        base = i * (8 * L)
        for u in range(8):
            tbl_v[pl.ds(base + u * L, L)] = zero
        return c

    lax.fori_loop(0, TPW // (8 * L), z_body, 0)

    din.wait()
    pltpu.sync_copy(bounce_v, pad_hbm.at[pl.ds(rbase, RCH)])
    din = pltpu.async_copy(inp_hbm.at[pl.ds(rbase + RCH, RCH)], bounce_v, sem_in)

    d1.wait()
    d2.wait()

    iota = lax.iota(jnp.int32, L)

    def u_body(i, c):
        o = org_v[pl.ds(i * L, L)]
        sp = samp_v[pl.ds(i * L, L)]
        local = o * S + sp - lo
        m = (local >= 0) & (local < TPW)
        val = i * L + iota + 1
        plsc.store_scatter(tbl_v, [local], val, mask=m)
        return c

    lax.fori_loop(0, NU // L, u_body, 0)

    din.wait()
    pltpu.sync_copy(bounce_v, pad_hbm.at[pl.ds(rbase + RCH, RCH)])

    zf = jnp.zeros((L,), jnp.float32)

    def zr_body(i, c):
        bounce_v[i >> 1, pl.ds((i & 1) * L, L)] = zf
        return c

    lax.fori_loop(0, ZPW * D // L, zr_body, 0)
    pltpu.sync_copy(bounce_v.at[pl.ds(0, ZPW)],
                    pad_hbm.at[pl.ds(NU + wid * ZPW, ZPW)])

    pltpu.sync_copy(tbl_v, tbl_hbm.at[pl.ds(lo, TPW)])


@jax.jit
def _build_table(org_ids, samp_ids, inp):
    return pl.kernel(
        _table_body,
        out_type=(jax.ShapeDtypeStruct((TPAD,), jnp.int32),
                  jax.ShapeDtypeStruct((NU + NZPAD, D), jnp.float32)),
        mesh=_mesh,
        compiler_params=_params,
        scratch_types=[
            pltpu.VMEM((NU,), jnp.int32),
            pltpu.VMEM((NU,), jnp.int32),
            pltpu.VMEM((TPW,), jnp.int32),
            pltpu.VMEM((RCH, D), jnp.float32),
            pltpu.SemaphoreType.DMA,
            pltpu.SemaphoreType.DMA,
        ],
    )(org_ids, samp_ids, inp)


def _gather_body(org_hbm, samp_hbm, tbl_hbm, inp_hbm, out_hbm,
                 org_v, samp_v, keys_v, jv_v, sel_v, rows_v, sem):
    wid = _worker_id()
    qbase = wid * QPW

    d1 = pltpu.async_copy(org_hbm, org_v, sem)
    d2 = pltpu.async_copy(samp_hbm, samp_v, sem)
    d1.wait()
    d2.wait()

    iota = lax.iota(jnp.int32, L)

    def k_body(i, c):
        q = qbase + i * L + iota
        o = q >> 12
        s = q & (NQ_S - 1)
        og = plsc.load_gather(org_v, [o])
        sg = plsc.load_gather(samp_v, [s])
        keys_v[i >> 3, pl.ds((i & 7) * L, L)] = og * S + sg
        return c

    lax.fori_loop(0, QPW // L, k_body, 0)

    descs = [
        pltpu.async_copy(tbl_hbm.at[keys_v.at[c]], jv_v.at[c], sem)
        for c in range(NCH)
    ]
    for d in descs:
        d.wait()

    def s_body(i, c):
        v = jv_v[i >> 3, pl.ds((i & 7) * L, L)]
        q = qbase + i * L + iota
        miss = ZROW + (q & (NZPAD - 1))
        sel_v[i >> 3, pl.ds((i & 7) * L, L)] = jnp.where(v > 0, v - 1, miss)
        return c

    lax.fori_loop(0, QPW // L, s_body, 0)

    descs = [
        pltpu.async_copy(inp_hbm.at[sel_v.at[c]],
                         rows_v.at[pl.ds(c * QCH, QCH)], sem)
        for c in range(NCH)
    ]
    for d in descs:
        d.wait()

    pltpu.sync_copy(rows_v, out_hbm.at[pl.ds(qbase, QPW)])


@jax.jit
def _join_gather(org_pad, samp_ids, table, inp_pad):
    return pl.kernel(
        _gather_body,
        out_type=jax.ShapeDtypeStruct((NQ, D), jnp.float32),
        mesh=_mesh,
        compiler_params=_params,
        scratch_types=[
            pltpu.VMEM((2 * L,), jnp.int32),
            pltpu.VMEM((NQ_S,), jnp.int32),
            pltpu.VMEM((NCH, QCH), jnp.int32),
            pltpu.VMEM((NCH, QCH), jnp.int32),
            pltpu.VMEM((NCH, QCH), jnp.int32),
            pltpu.VMEM((QPW, D), jnp.float32),
            pltpu.SemaphoreType.DMA,
        ],
    )(org_pad, samp_ids, table, inp_pad)


def kernel(buffer, sample_id, organization_id, input, get_sample_id, get_org_id):
    del buffer
    org_ids = organization_id.astype(jnp.int32)
    samp_ids = sample_id.astype(jnp.int32)
    org_pad = jnp.concatenate(
        [get_org_id.astype(jnp.int32), jnp.zeros((2 * L - NQ_O,), jnp.int32)])
    table, inp_pad = _build_table(org_ids, samp_ids, input)
    out = _join_gather(org_pad, get_sample_id.astype(jnp.int32), table, inp_pad)
    return out.reshape(NQ_O, NQ_S, D)


# speedup vs baseline: 1.0748x; 1.0727x over previous
import functools

import jax
import jax.numpy as jnp
from jax import lax
from jax.experimental import pallas as pl
from jax.experimental.pallas import tpu as pltpu
from jax.experimental.pallas import tpu_sc as plsc

NC, NS = 2, 16
NW = NC * NS
L = 16

O, S, D = 26, 100000, 32
NU = 16384
NQ_O, NQ_S = 26, 4096
NQ = NQ_O * NQ_S
TPW = 81920
TPAD = NW * TPW
QPW = NQ // NW
QCH = 128
NCH = QPW // QCH
ZROW = NU
NZPAD = 2048

_mesh = plsc.VectorSubcoreMesh(core_axis_name="c", subcore_axis_name="s")
_params = pltpu.CompilerParams(needs_layout_passes=False,
                               use_tc_tiling_on_sc=False)


def _worker_id():
    return lax.axis_index("s") * NC + lax.axis_index("c")


def _table_body(org_hbm, samp_hbm, tbl_hbm, org_v, samp_v, tbl_v, sem):
    wid = _worker_id()
    lo = wid * TPW

    d1 = pltpu.async_copy(org_hbm, org_v, sem)
    d2 = pltpu.async_copy(samp_hbm, samp_v, sem)

    zero = jnp.zeros((L,), jnp.int32)

    def z_body(i, c):
        base = i * (8 * L)
        for u in range(8):
            tbl_v[pl.ds(base + u * L, L)] = zero
        return c

    lax.fori_loop(0, TPW // (8 * L), z_body, 0)
    d1.wait()
    d2.wait()

    iota = lax.iota(jnp.int32, L)

    def u_body(i, c):
        o = org_v[pl.ds(i * L, L)]
        sp = samp_v[pl.ds(i * L, L)]
        local = o * S + sp - lo
        m = (local >= 0) & (local < TPW)
        val = i * L + iota + 1
        plsc.store_scatter(tbl_v, [local], val, mask=m)
        return c

    lax.fori_loop(0, NU // L, u_body, 0)
    pltpu.sync_copy(tbl_v, tbl_hbm.at[pl.ds(lo, TPW)])


@jax.jit
def _build_table(org_ids, samp_ids):
    return pl.kernel(
        _table_body,
        out_type=jax.ShapeDtypeStruct((TPAD,), jnp.int32),
        mesh=_mesh,
        compiler_params=_params,
        scratch_types=[
            pltpu.VMEM((NU,), jnp.int32),
            pltpu.VMEM((NU,), jnp.int32),
            pltpu.VMEM((TPW,), jnp.int32),
            pltpu.SemaphoreType.DMA,
        ],
    )(org_ids, samp_ids)


def _gather_body(org_hbm, samp_hbm, tbl_hbm, inp_hbm, out_hbm,
                 org_v, samp_v, keys_v, jv_v, sel_v, rows_v, sem):
    wid = _worker_id()
    sbase = wid * QCH

    d1 = pltpu.async_copy(org_hbm, org_v, sem)
    d2 = pltpu.async_copy(samp_hbm.at[pl.ds(sbase, QCH)], samp_v, sem)
    d1.wait()
    d2.wait()

    iota = lax.iota(jnp.int32, L)

    def k_body(i, c):
        o = i >> 3
        u = i & 7
        og = plsc.load_gather(org_v, [jnp.zeros((L,), jnp.int32) + o])
        sg = samp_v[pl.ds(u * L, L)]
        keys_v[o, pl.ds(u * L, L)] = og * S + sg
        return c

    lax.fori_loop(0, NCH * 8, k_body, 0)

    descs = [
        pltpu.async_copy(tbl_hbm.at[keys_v.at[c]], jv_v.at[c], sem)
        for c in range(NCH)
    ]
    for d in descs:
        d.wait()

    def s_body(i, c):
        v = jv_v[i >> 3, pl.ds((i & 7) * L, L)]
        spread = (i * L + sbase) + iota
        miss = ZROW + (spread & (NZPAD - 1))
        sel_v[i >> 3, pl.ds((i & 7) * L, L)] = jnp.where(v > 0, v - 1, miss)
        return c

    lax.fori_loop(0, NCH * 8, s_body, 0)

    descs = [
        pltpu.async_copy(inp_hbm.at[sel_v.at[c]],
                         rows_v.at[pl.ds(c * QCH, QCH)], sem)
        for c in range(NCH)
    ]
    for d in descs:
        d.wait()

    descs = [
        pltpu.async_copy(rows_v.at[pl.ds(c * QCH, QCH)],
                         out_hbm.at[c].at[pl.ds(sbase, QCH)], sem)
        for c in range(NCH)
    ]
    for d in descs:
        d.wait()


@jax.jit
def _join_gather(org_pad, samp_ids, table, inp_pad):
    return pl.kernel(
        _gather_body,
        out_type=jax.ShapeDtypeStruct((NQ_O, NQ_S, D), jnp.float32),
        mesh=_mesh,
        compiler_params=_params,
        scratch_types=[
            pltpu.VMEM((2 * L,), jnp.int32),
            pltpu.VMEM((QCH,), jnp.int32),
            pltpu.VMEM((NCH, QCH), jnp.int32),
            pltpu.VMEM((NCH, QCH), jnp.int32),
            pltpu.VMEM((NCH, QCH), jnp.int32),
            pltpu.VMEM((QPW, D), jnp.float32),
            pltpu.SemaphoreType.DMA,
        ],
    )(org_pad, samp_ids, table, inp_pad)


def kernel(buffer, sample_id, organization_id, input, get_sample_id, get_org_id):
    del buffer
    org_ids = organization_id.astype(jnp.int32)
    samp_ids = sample_id.astype(jnp.int32)
    inp_pad = jnp.concatenate(
        [input, jnp.zeros((NZPAD, D), input.dtype)], axis=0)
    org_pad = jnp.concatenate(
        [get_org_id.astype(jnp.int32), jnp.zeros((2 * L - NQ_O,), jnp.int32)])
    table = _build_table(org_ids, samp_ids)
    return _join_gather(org_pad, get_sample_id.astype(jnp.int32), table, inp_pad)
